# Initial kernel scaffold; baseline (speedup 1.0000x reference)
import jax
import jax.numpy as jnp
from jax.experimental import pallas as pl


def kernel(adj_indices, adj_values, user_emb, item_emb, brand_emb):
    raise NotImplementedError("write your pallas kernel here")


# speedup vs baseline: 6.7236x; 6.7236x over previous
import functools

import jax
import jax.numpy as jnp
from jax import lax
from jax.experimental import pallas as pl
from jax.experimental.pallas import tpu as pltpu
from jax.experimental.pallas import tpu_sc as plsc

_NUM_USERS = 25000
_NUM_ITEMS = 22000
_NNODES = 50000
_NP = 51200
_D = 64
_DH = 32
_NLAYERS = 3
_E = 800000
_NC, _NS = 2, 16
_CHUNK = 128
_K = 4
_GROUPS = 98
_CPT = _K * _GROUPS
_E_PAD = _NS * _CPT * _CHUNK
_RPT = _NP // _NS
_ZROWS = 160
_MROWS = 128


def _body(ego, src, dst, val, t1, t2, t3, fin,
          acc, srcb, dstb, valb, rows, zbuf, sem):
    c = lax.axis_index("c")
    s = lax.axis_index("s")
    zeros16 = jnp.zeros((16,), jnp.float32)

    def _zb(r, carry):
        zbuf[r, pl.ds(0, 16)] = zeros16
        zbuf[r, pl.ds(16, 16)] = zeros16
        return carry
    lax.fori_loop(0, _ZROWS, _zb, 0)

    r0 = s * _RPT
    tables = [ego, t1, t2, t3]
    for layer in range(_NLAYERS):
        src_table = tables[layer]
        out_table = tables[layer + 1]

        for i in range(_RPT // _ZROWS):
            pltpu.sync_copy(zbuf, acc.at[pl.ds(r0 + i * _ZROWS, _ZROWS)])
        plsc.subcore_barrier()

        def _group(g, carry, src_table=src_table):
            base = s * _CPT + g * _K
            pltpu.sync_copy(src.at[pl.ds(base, _K)], srcb)
            pltpu.sync_copy(dst.at[pl.ds(base, _K)], dstb)
            pltpu.sync_copy(val.at[pl.ds(base, _K)], valb)
            cps = [pltpu.async_copy(src_table.at[c].at[srcb.at[j]],
                                    rows.at[j], sem)
                   for j in range(_K)]
            for cp in cps:
                cp.wait()
            for j in range(_K):
                def _edge16(i16, carry2, j=j):
                    vv = valb[j, pl.ds(i16 * 16, 16)]
                    for e in range(16):
                        i = i16 * 16 + e
                        v = vv[e]
                        a = rows[j, i, pl.ds(0, 16)]
                        b = rows[j, i, pl.ds(16, 16)]
                        rows[j, i, pl.ds(0, 16)] = a * v
                        rows[j, i, pl.ds(16, 16)] = b * v
                    return carry2
                lax.fori_loop(0, _CHUNK // 16, _edge16, 0)
            for j in range(_K):
                pltpu.sync_copy(rows.at[j], acc.at[dstb.at[j]], add=True)
            return carry
        lax.fori_loop(0, _GROUPS, _group, 0)

        plsc.subcore_barrier()
        pltpu.sync_copy(acc.at[pl.ds(r0, _RPT)],
                        out_table.at[c, pl.ds(r0, _RPT)])
        plsc.subcore_barrier()

    def _mean_chunk(m, carry):
        r = r0 + m * _MROWS
        for t in range(4):
            pltpu.sync_copy(tables[t].at[c, pl.ds(r, _MROWS)],
                            rows.at[t, pl.ds(0, _MROWS)])
        def _row(i, carry2):
            for h in range(2):
                x0 = rows[0, i, pl.ds(h * 16, 16)]
                x1 = rows[1, i, pl.ds(h * 16, 16)]
                x2 = rows[2, i, pl.ds(h * 16, 16)]
                x3 = rows[3, i, pl.ds(h * 16, 16)]
                rows[0, i, pl.ds(h * 16, 16)] = (x0 + x1 + x2 + x3) * 0.25
            return carry2
        lax.fori_loop(0, _MROWS, _row, 0)
        pltpu.sync_copy(rows.at[0, pl.ds(0, _MROWS)],
                        fin.at[c, pl.ds(r, _MROWS)])
        return carry
    lax.fori_loop(0, _RPT // _MROWS, _mean_chunk, 0)


_TBL = jax.ShapeDtypeStruct((_NC, _NP, _DH), jnp.float32)

_prop = pl.kernel(
    _body,
    out_type=(_TBL, _TBL, _TBL, _TBL),
    mesh=plsc.VectorSubcoreMesh(core_axis_name="c", subcore_axis_name="s"),
    compiler_params=pltpu.CompilerParams(use_tc_tiling_on_sc=False),
    scratch_types=(
        pltpu.VMEM_SHARED((_NP, _DH), jnp.float32),
        pltpu.VMEM((_K, _CHUNK), jnp.int32),
        pltpu.VMEM((_K, _CHUNK), jnp.int32),
        pltpu.VMEM((_K, _CHUNK), jnp.float32),
        pltpu.VMEM((_K, _CHUNK, _DH), jnp.float32),
        pltpu.VMEM((_ZROWS, _DH), jnp.float32),
        pltpu.SemaphoreType.DMA,
    ),
)


def kernel(adj_indices, adj_values, user_emb, item_emb, brand_emb):
    ego0 = jnp.concatenate([
        user_emb, item_emb, brand_emb,
        jnp.zeros((_NP - _NNODES, _D), jnp.float32)], axis=0)
    egoh = ego0.reshape(_NP, _NC, _DH).transpose(1, 0, 2)
    pad = _E_PAD - _E
    src = jnp.concatenate([adj_indices[1], jnp.zeros((pad,), jnp.int32)])
    dst = jnp.concatenate([adj_indices[0], jnp.zeros((pad,), jnp.int32)])
    val = jnp.concatenate([adj_values, jnp.zeros((pad,), jnp.float32)])
    src2 = src.reshape(-1, _CHUNK)
    dst2 = dst.reshape(-1, _CHUNK)
    val2 = val.reshape(-1, _CHUNK)
    _t1, _t2, _t3, fin = _prop(egoh, src2, dst2, val2)
    final = fin.transpose(1, 0, 2).reshape(_NP, _D)
    return final[:_NUM_USERS], final[_NUM_USERS:_NUM_USERS + _NUM_ITEMS]
